# baseline (speedup 1.0000x reference)
import jax
import jax.numpy as jnp
from jax.experimental import pallas as pl

B = 16384
LT = 50
LP = 200
V = 20
D = 64
H = 128
CB = 2048
LTP = 64
LPP = 208


def _counts_t(tok_ref, lp):
    tok = tok_ref[:, :]
    ntile = lp // 16
    one = jnp.ones((), jnp.bfloat16)
    zero = jnp.zeros((), jnp.bfloat16)
    rows = []
    for v in range(V):
        m = jnp.where(tok == jnp.bfloat16(v), one, zero)
        acc = m[0:16]
        for t in range(1, ntile):
            acc = acc + m[16 * t:16 * (t + 1)]
        rows.append(jnp.sum(acc.astype(jnp.float32), axis=0, keepdims=True))
    return jnp.concatenate(rows, axis=0)


def _body(tcr_ref, lt_ref, pmhc_ref, lp_ref, embed_ref, w1_ref, b1_ref,
          w2_ref, b2_ref, out_ref):
    embed = embed_ref[:, :]
    w1 = w1_ref[:, :]
    dn = (((1,), (1,)), ((), ()))
    e1a = jax.lax.dot_general(embed, w1[:, :D], dn,
                              preferred_element_type=jnp.float32)
    e1b = jax.lax.dot_general(embed, w1[:, D:], dn,
                              preferred_element_type=jnp.float32)

    ct = _counts_t(tcr_ref, LTP) * (1.0 / lt_ref[:, :])
    cp = _counts_t(pmhc_ref, LPP) * (1.0 / lp_ref[:, :])

    dnt = (((0,), (0,)), ((), ()))
    h = (jax.lax.dot_general(ct, e1a, dnt, preferred_element_type=jnp.float32)
         + jax.lax.dot_general(cp, e1b, dnt,
                               preferred_element_type=jnp.float32)
         + b1_ref[:, :])
    h = jnp.maximum(h, 0.0)
    out_ref[:, :] = (jnp.sum(h * w2_ref[:, :], axis=1, keepdims=True)
                     + b2_ref[:, :])


def kernel(tcr, tcr_len, pmhc, pmhc_len, embed, W1, b1, W2, b2):
    tcr_t = jnp.pad(tcr.T.astype(jnp.bfloat16), ((0, LTP - LT), (0, 0)),
                    constant_values=jnp.bfloat16(255))
    pmhc_t = jnp.pad(pmhc.T.astype(jnp.bfloat16), ((0, LPP - LP), (0, 0)),
                     constant_values=jnp.bfloat16(255))
    grid = (B // CB,)
    out = pl.pallas_call(
        _body,
        grid=grid,
        in_specs=[
            pl.BlockSpec((LTP, CB), lambda i: (0, i)),
            pl.BlockSpec((1, CB), lambda i: (0, i)),
            pl.BlockSpec((LPP, CB), lambda i: (0, i)),
            pl.BlockSpec((1, CB), lambda i: (0, i)),
            pl.BlockSpec((V, D), lambda i: (0, 0)),
            pl.BlockSpec((H, 2 * D), lambda i: (0, 0)),
            pl.BlockSpec((1, H), lambda i: (0, 0)),
            pl.BlockSpec((1, H), lambda i: (0, 0)),
            pl.BlockSpec((1, 1), lambda i: (0, 0)),
        ],
        out_specs=pl.BlockSpec((CB, 1), lambda i: (i, 0)),
        out_shape=jax.ShapeDtypeStruct((B, 1), jnp.float32),
    )(tcr_t, tcr_len.reshape(1, B), pmhc_t, pmhc_len.reshape(1, B),
      embed, W1, b1.reshape(1, H), W2, b2.reshape(1, 1))
    return out[:, 0]
